# Initial kernel scaffold; baseline (speedup 1.0000x reference)
import jax
import jax.numpy as jnp
from jax.experimental import pallas as pl


def kernel(features, nodes, adj_lists, mask, unique_nodes_list, weight):
    raise NotImplementedError("write your pallas kernel here")


# speedup vs baseline: 1.5127x; 1.5127x over previous
import functools

import jax
import jax.numpy as jnp
from jax import lax
from jax.experimental import pallas as pl
from jax.experimental.pallas import tpu as pltpu
from jax.experimental.pallas import tpu_sc as plsc

_FEATURE_DIM = 128
_EMBED_DIM = 128
_B = 4096
_U = 16384
_ALPHA = 0.2

_IDX_CHUNK = 128


def _make_sc_gather(num_rows, feature_dim):
    info = plsc.get_sparse_core_info()
    nw = info.num_cores * info.num_subcores
    rows_per_w = num_rows // nw
    n_chunks = rows_per_w // _IDX_CHUNK
    chunks_per_w = n_chunks

    mesh = plsc.VectorSubcoreMesh(core_axis_name="c", subcore_axis_name="s")

    @functools.partial(
        pl.kernel,
        mesh=mesh,
        out_type=jax.ShapeDtypeStruct((num_rows, feature_dim), jnp.float32),
        scratch_types=[
            pltpu.VMEM((chunks_per_w, _IDX_CHUNK), jnp.int32),
            pltpu.VMEM((rows_per_w, feature_dim), jnp.float32),
            pltpu.SemaphoreType.DMA,
        ],
    )
    def gather_kernel(table_hbm, idx_hbm, out_hbm, idx_v, rows_v, sem):
        wid = lax.axis_index("s") * info.num_cores + lax.axis_index("c")
        base = wid * rows_per_w
        pltpu.sync_copy(idx_hbm.at[pl.ds(wid * chunks_per_w, chunks_per_w)],
                        idx_v)
        copies = []
        for j in range(n_chunks):
            copies.append(pltpu.async_copy(
                table_hbm.at[idx_v.at[j]],
                rows_v.at[pl.ds(j * _IDX_CHUNK, _IDX_CHUNK)],
                sem,
            ))
        for c in copies:
            c.wait()
        pltpu.sync_copy(rows_v, out_hbm.at[pl.ds(base, rows_per_w)])

    return gather_kernel


def _mm_body(mask_ref, embed_ref, wt_ref, out_ref):
    m = mask_ref[...].astype(jnp.bfloat16)
    e = embed_ref[...].astype(jnp.bfloat16)
    acc = lax.dot_general(m, e, (((1,), (0,)), ((), ())),
                          preferred_element_type=jnp.float32)
    out = lax.dot_general(acc, wt_ref[...], (((1,), (0,)), ((), ())),
                          preferred_element_type=jnp.float32)
    out_ref[...] = jnp.where(out >= 0, out, _ALPHA * out)


def _make_tc_matmul(b, u, d, bm):
    grid = (b // bm,)
    return pl.pallas_call(
        _mm_body,
        grid=grid,
        in_specs=[
            pl.BlockSpec((bm, u), lambda i: (i, 0)),
            pl.BlockSpec((u, d), lambda i: (0, 0)),
            pl.BlockSpec((d, d), lambda i: (0, 0)),
        ],
        out_specs=pl.BlockSpec((bm, d), lambda i: (i, 0)),
        out_shape=jax.ShapeDtypeStruct((b, d), jnp.float32),
    )


def kernel(features, nodes, adj_lists, mask, unique_nodes_list, weight):
    del nodes, adj_lists
    idx = unique_nodes_list.astype(jnp.int32).reshape(_U // _IDX_CHUNK,
                                                      _IDX_CHUNK)
    embed = _make_sc_gather(_U, _FEATURE_DIM)(features, idx)
    wt = weight.T
    return _make_tc_matmul(_B, _U, _EMBED_DIM, 128)(mask, embed, wt)
